# Initial kernel scaffold; baseline (speedup 1.0000x reference)
import jax
import jax.numpy as jnp
from jax.experimental import pallas as pl


def kernel(text, seq_len, text_embed_weight):
    raise NotImplementedError("write your pallas kernel here")


# speedup vs baseline: 2.6504x; 2.6504x over previous
import functools

import jax
import jax.numpy as jnp
from jax import lax
from jax.experimental import pallas as pl
from jax.experimental.pallas import tpu as pltpu
from jax.experimental.pallas import tpu_sc as plsc

BATCH = 32
NT = 2048
TEXT_DIM = 512
LANES = 16
NUM_CORES = 2
NUM_SUBCORES = 16
CHUNK = 64
NCHUNK = NT // CHUNK


def _sc_embed(text, seq_len_vec, table):
    mesh = plsc.VectorSubcoreMesh(
        core_axis_name="c", subcore_axis_name="s",
        num_cores=NUM_CORES, num_subcores=NUM_SUBCORES,
    )

    @functools.partial(
        pl.kernel,
        out_type=jax.ShapeDtypeStruct((BATCH, NT, TEXT_DIM), jnp.float32),
        mesh=mesh,
        scratch_types=[
            pltpu.VMEM((NT,), jnp.int32),
            pltpu.VMEM((LANES,), jnp.int32),
            pltpu.VMEM((CHUNK, TEXT_DIM), jnp.float32),
            pltpu.VMEM((CHUNK, TEXT_DIM), jnp.float32),
            pltpu.SemaphoreType.DMA,
            pltpu.SemaphoreType.DMA,
        ],
    )
    def k(text_hbm, slv_hbm, table_hbm, out_hbm,
          idx_v, slv_v, rows0, rows1, sem0, sem1):
        wid = lax.axis_index("s") * NUM_CORES + lax.axis_index("c")

        pltpu.sync_copy(text_hbm.at[wid], idx_v)
        pltpu.sync_copy(slv_hbm, slv_v)
        sl = slv_v[...]

        def prep(i, carry):
            base = pl.multiple_of(i * LANES, LANES)
            v = idx_v[pl.ds(base, LANES)]
            col = lax.iota(jnp.int32, LANES) + i * LANES
            idx_v[pl.ds(base, LANES)] = jnp.where(col < sl, v + 1, 0)
            return carry

        lax.fori_loop(0, NT // LANES, prep, 0)

        def gather(c, rows, sem):
            src = idx_v.at[pl.ds(pl.multiple_of(c * CHUNK, CHUNK), CHUNK)]
            return pltpu.async_copy(table_hbm.at[src], rows, sem)

        def writeback(c, rows):
            dst = out_hbm.at[wid, pl.ds(pl.multiple_of(c * CHUNK, CHUNK), CHUNK)]
            pltpu.sync_copy(rows, dst)

        gather(0, rows0, sem0)

        def wait_gather(rows, sem):
            pltpu.make_async_copy(table_hbm.at[idx_v.at[pl.ds(0, CHUNK)]],
                                  rows, sem).wait()

        def step(g, carry):
            c0 = g * 2
            wait_gather(rows0, sem0)
            d1 = gather(c0 + 1, rows1, sem1)
            writeback(c0, rows0)
            d1.wait()

            @pl.when(g + 1 < NCHUNK // 2)
            def _():
                gather(c0 + 2, rows0, sem0)

            writeback(c0 + 1, rows1)
            return carry

        lax.fori_loop(0, NCHUNK // 2, step, 0)

    return k(text, seq_len_vec, table)


def kernel(text, seq_len, text_embed_weight):
    text_i32 = text.astype(jnp.int32)
    slv = jnp.full((LANES,), seq_len, dtype=jnp.int32)
    return _sc_embed(text_i32, slv, text_embed_weight)
